# Initial kernel scaffold; baseline (speedup 1.0000x reference)
import jax
import jax.numpy as jnp
from jax.experimental import pallas as pl


def kernel(x, edge_index, edge_weight, W_xz, b_xz, W_hz, b_hz, W_xr, b_xr, W_hr, b_hr, W_xh, b_xh, W_hh, b_hh, W_lin, b_lin):
    raise NotImplementedError("write your pallas kernel here")


# speedup vs baseline: 1.0213x; 1.0213x over previous
import jax
import jax.numpy as jnp
from jax.experimental import pallas as pl
from jax.experimental.pallas import tpu as pltpu

_F = 128
_OUT = 64


def _body(x_ref, wa_ref, ba_ref, wl_ref, bl_ref, out_ref):
    a = jnp.dot(x_ref[:], wa_ref[:], preferred_element_type=jnp.float32)
    z = jax.nn.sigmoid(a[:, :_F] + ba_ref[0, :_F])
    t = jnp.tanh(a[:, _F:] + ba_ref[0, _F:])
    h = jnp.maximum((1.0 - z) * t, 0.0)
    out_ref[:] = (
        jnp.dot(h, wl_ref[:], preferred_element_type=jnp.float32) + bl_ref[0]
    )


def kernel(x, edge_index, edge_weight, W_xz, b_xz, W_hz, b_hz, W_xr, b_xr,
           W_hr, b_hr, W_xh, b_xh, W_hh, b_hh, W_lin, b_lin):
    del edge_index, edge_weight, W_hz, W_xr, b_xr, W_hr, b_hr, W_hh
    n = x.shape[0]
    block = 1000
    grid = (n // block,)

    wa = jnp.concatenate([W_xz, W_xh], axis=1)
    ba = jnp.concatenate([b_xz + b_hz, b_xh + b_hh]).reshape(1, 2 * _F)
    bl = b_lin.reshape(1, _OUT)

    out = pl.pallas_call(
        _body,
        grid=grid,
        in_specs=[
            pl.BlockSpec((block, _F), lambda i: (i, 0)),
            pl.BlockSpec((_F, 2 * _F), lambda i: (0, 0)),
            pl.BlockSpec((1, 2 * _F), lambda i: (0, 0)),
            pl.BlockSpec((_F, _OUT), lambda i: (0, 0)),
            pl.BlockSpec((1, _OUT), lambda i: (0, 0)),
        ],
        out_specs=pl.BlockSpec((block, _OUT), lambda i: (i, 0)),
        out_shape=jax.ShapeDtypeStruct((n, _OUT), jnp.float32),
        compiler_params=pltpu.CompilerParams(
            dimension_semantics=("parallel",),
        ),
    )(x, wa, ba, W_lin, bl)
    return (out,)


# speedup vs baseline: 1.1691x; 1.1447x over previous
import jax
import jax.numpy as jnp
from jax.experimental import pallas as pl
from jax.experimental.pallas import tpu as pltpu

_F = 128
_OUT = 64


def _body(x_ref, wz_ref, wh_ref, wl_ref, bxz_ref, bhz_ref, bxh_ref, bhh_ref,
          bl_ref, out_ref):
    xb = x_ref[:]
    az = jnp.dot(xb, wz_ref[:], preferred_element_type=jnp.float32)
    ah = jnp.dot(xb, wh_ref[:], preferred_element_type=jnp.float32)
    z = jax.nn.sigmoid(az + (bxz_ref[0] + bhz_ref[0]))
    t = jnp.tanh(ah + (bxh_ref[0] + bhh_ref[0]))
    h = jnp.maximum((1.0 - z) * t, 0.0)
    out_ref[:] = (
        jnp.dot(h, wl_ref[:], preferred_element_type=jnp.float32) + bl_ref[0]
    )


def kernel(x, edge_index, edge_weight, W_xz, b_xz, W_hz, b_hz, W_xr, b_xr,
           W_hr, b_hr, W_xh, b_xh, W_hh, b_hh, W_lin, b_lin):
    del edge_index, edge_weight, W_hz, W_xr, b_xr, W_hr, b_hr, W_hh
    n = x.shape[0]
    block = 1000
    grid = (n // block,)

    bxz = b_xz.reshape(1, _F)
    bhz = b_hz.reshape(1, _F)
    bxh = b_xh.reshape(1, _F)
    bhh = b_hh.reshape(1, _F)
    bl = b_lin.reshape(1, _OUT)

    full = lambda i: (0, 0)
    out = pl.pallas_call(
        _body,
        grid=grid,
        in_specs=[
            pl.BlockSpec((block, _F), lambda i: (i, 0)),
            pl.BlockSpec((_F, _F), full),
            pl.BlockSpec((_F, _F), full),
            pl.BlockSpec((_F, _OUT), full),
            pl.BlockSpec((1, _F), full),
            pl.BlockSpec((1, _F), full),
            pl.BlockSpec((1, _F), full),
            pl.BlockSpec((1, _F), full),
            pl.BlockSpec((1, _OUT), full),
        ],
        out_specs=pl.BlockSpec((block, _OUT), lambda i: (i, 0)),
        out_shape=jax.ShapeDtypeStruct((n, _OUT), jnp.float32),
        compiler_params=pltpu.CompilerParams(
            dimension_semantics=("parallel",),
        ),
    )(x, W_xz, W_xh, W_lin, bxz, bhz, bxh, bhh, bl)
    return (out,)


# speedup vs baseline: 1.4156x; 1.2109x over previous
import jax
import jax.numpy as jnp
from jax.experimental import pallas as pl
from jax.experimental.pallas import tpu as pltpu

_F = 128
_OUT = 64


def _body(x_ref, wz_ref, wh_ref, wl_ref, bxz_ref, bhz_ref, bxh_ref, bhh_ref,
          bl_ref, out_ref):
    xb = x_ref[:]
    az = jnp.dot(xb, wz_ref[:], preferred_element_type=jnp.float32)
    ah = jnp.dot(xb, wh_ref[:], preferred_element_type=jnp.float32)
    z = jax.nn.sigmoid(az + (bxz_ref[0] + bhz_ref[0]))
    t = jnp.tanh(ah + (bxh_ref[0] + bhh_ref[0]))
    h = jnp.maximum((1.0 - z) * t, 0.0)
    out_ref[:] = (
        jnp.dot(h, wl_ref[:], preferred_element_type=jnp.float32) + bl_ref[0]
    )


def kernel(x, edge_index, edge_weight, W_xz, b_xz, W_hz, b_hz, W_xr, b_xr,
           W_hr, b_hr, W_xh, b_xh, W_hh, b_hh, W_lin, b_lin):
    del edge_index, edge_weight, W_hz, W_xr, b_xr, W_hr, b_hr, W_hh
    n = x.shape[0]
    block = 2000
    grid = (n // block,)

    bxz = b_xz.reshape(1, _F)
    bhz = b_hz.reshape(1, _F)
    bxh = b_xh.reshape(1, _F)
    bhh = b_hh.reshape(1, _F)
    bl = b_lin.reshape(1, _OUT)

    full = lambda i: (0, 0)
    out = pl.pallas_call(
        _body,
        grid=grid,
        in_specs=[
            pl.BlockSpec((block, _F), lambda i: (i, 0)),
            pl.BlockSpec((_F, _F), full),
            pl.BlockSpec((_F, _F), full),
            pl.BlockSpec((_F, _OUT), full),
            pl.BlockSpec((1, _F), full),
            pl.BlockSpec((1, _F), full),
            pl.BlockSpec((1, _F), full),
            pl.BlockSpec((1, _F), full),
            pl.BlockSpec((1, _OUT), full),
        ],
        out_specs=pl.BlockSpec((block, _OUT), lambda i: (i, 0)),
        out_shape=jax.ShapeDtypeStruct((n, _OUT), jnp.float32),
        compiler_params=pltpu.CompilerParams(
            dimension_semantics=("parallel",),
        ),
    )(x, W_xz, W_xh, W_lin, bxz, bhz, bxh, bhh, bl)
    return (out,)


# speedup vs baseline: 1.4205x; 1.0034x over previous
import jax
import jax.numpy as jnp
from jax.experimental import pallas as pl
from jax.experimental.pallas import tpu as pltpu

_F = 128
_OUT = 64


def _body(x_ref, wz_ref, wh_ref, wl_ref, bxz_ref, bhz_ref, bxh_ref, bhh_ref,
          bl_ref, out_ref):
    xb = x_ref[:]
    az = jnp.dot(xb, wz_ref[:], preferred_element_type=jnp.float32)
    ah = jnp.dot(xb, wh_ref[:], preferred_element_type=jnp.float32)
    z = jax.nn.sigmoid(az + (bxz_ref[0] + bhz_ref[0]))
    t = jnp.tanh(ah + (bxh_ref[0] + bhh_ref[0]))
    h = jnp.maximum((1.0 - z) * t, 0.0)
    out_ref[:] = (
        jnp.dot(h, wl_ref[:], preferred_element_type=jnp.float32) + bl_ref[0]
    )


def kernel(x, edge_index, edge_weight, W_xz, b_xz, W_hz, b_hz, W_xr, b_xr,
           W_hr, b_hr, W_xh, b_xh, W_hh, b_hh, W_lin, b_lin):
    del edge_index, edge_weight, W_hz, W_xr, b_xr, W_hr, b_hr, W_hh
    n = x.shape[0]
    block = 5000
    grid = (n // block,)

    bxz = b_xz.reshape(1, _F)
    bhz = b_hz.reshape(1, _F)
    bxh = b_xh.reshape(1, _F)
    bhh = b_hh.reshape(1, _F)
    bl = b_lin.reshape(1, _OUT)

    full = lambda i: (0, 0)
    out = pl.pallas_call(
        _body,
        grid=grid,
        in_specs=[
            pl.BlockSpec((block, _F), lambda i: (i, 0)),
            pl.BlockSpec((_F, _F), full),
            pl.BlockSpec((_F, _F), full),
            pl.BlockSpec((_F, _OUT), full),
            pl.BlockSpec((1, _F), full),
            pl.BlockSpec((1, _F), full),
            pl.BlockSpec((1, _F), full),
            pl.BlockSpec((1, _F), full),
            pl.BlockSpec((1, _OUT), full),
        ],
        out_specs=pl.BlockSpec((block, _OUT), lambda i: (i, 0)),
        out_shape=jax.ShapeDtypeStruct((n, _OUT), jnp.float32),
        compiler_params=pltpu.CompilerParams(
            dimension_semantics=("parallel",),
        ),
    )(x, W_xz, W_xh, W_lin, bxz, bhz, bxh, bhh, bl)
    return (out,)


# speedup vs baseline: 1.4699x; 1.0348x over previous
import jax
import jax.numpy as jnp
from jax.experimental import pallas as pl
from jax.experimental.pallas import tpu as pltpu

_F = 128
_OUT = 64


def _body(x_ref, wz_ref, wh_ref, wl_ref, bxz_ref, bhz_ref, bxh_ref, bhh_ref,
          bl_ref, out_ref):
    xb = x_ref[:]
    az = jnp.dot(xb, wz_ref[:], preferred_element_type=jnp.float32)
    ah = jnp.dot(xb, wh_ref[:], preferred_element_type=jnp.float32)
    z = jax.nn.sigmoid(az + (bxz_ref[0] + bhz_ref[0]))
    t = jnp.tanh(ah + (bxh_ref[0] + bhh_ref[0]))
    h = jnp.maximum((1.0 - z) * t, 0.0)
    out_ref[:] = (
        jnp.dot(h, wl_ref[:], preferred_element_type=jnp.float32) + bl_ref[0]
    )


def kernel(x, edge_index, edge_weight, W_xz, b_xz, W_hz, b_hz, W_xr, b_xr,
           W_hr, b_hr, W_xh, b_xh, W_hh, b_hh, W_lin, b_lin):
    del edge_index, edge_weight, W_hz, W_xr, b_xr, W_hr, b_hr, W_hh
    n = x.shape[0]
    block = n
    grid = (n // block,)

    bxz = b_xz.reshape(1, _F)
    bhz = b_hz.reshape(1, _F)
    bxh = b_xh.reshape(1, _F)
    bhh = b_hh.reshape(1, _F)
    bl = b_lin.reshape(1, _OUT)

    full = lambda i: (0, 0)
    out = pl.pallas_call(
        _body,
        grid=grid,
        in_specs=[
            pl.BlockSpec((block, _F), lambda i: (i, 0)),
            pl.BlockSpec((_F, _F), full),
            pl.BlockSpec((_F, _F), full),
            pl.BlockSpec((_F, _OUT), full),
            pl.BlockSpec((1, _F), full),
            pl.BlockSpec((1, _F), full),
            pl.BlockSpec((1, _F), full),
            pl.BlockSpec((1, _F), full),
            pl.BlockSpec((1, _OUT), full),
        ],
        out_specs=pl.BlockSpec((block, _OUT), lambda i: (i, 0)),
        out_shape=jax.ShapeDtypeStruct((n, _OUT), jnp.float32),
        compiler_params=pltpu.CompilerParams(
            dimension_semantics=("parallel",),
        ),
    )(x, W_xz, W_xh, W_lin, bxz, bhz, bxh, bhh, bl)
    return (out,)
